# baseline (speedup 1.0000x reference)
import jax
import jax.numpy as jnp
from jax.experimental import pallas as pl


def _body(row_ref, col_ref, out_ref):
    h, out_n = row_ref.shape
    top = jnp.broadcast_to(row_ref[...][None, :, :], (h, h, out_n))
    bot = jnp.broadcast_to(col_ref[...][:, None, :], (h, h, out_n))
    slab = jnp.concatenate([top, bot], axis=2)
    out_ref[...] = jnp.broadcast_to(slab[None], out_ref.shape)


def kernel(x, row_table, col_table):
    bs, _, h, w = x.shape
    out_n = row_table.shape[1]
    c_total = 2 * out_n
    bblk = 1

    y = pl.pallas_call(
        _body,
        grid=(bs // bblk,),
        in_specs=[
            pl.BlockSpec((h, out_n), lambda b: (0, 0)),
            pl.BlockSpec((w, out_n), lambda b: (0, 0)),
        ],
        out_specs=pl.BlockSpec((bblk, h, w, c_total), lambda b: (b, 0, 0, 0)),
        out_shape=jax.ShapeDtypeStruct((bs, h, w, c_total), jnp.float32),
    )(row_table, col_table)
    return jnp.transpose(y, (0, 3, 1, 2))


# speedup vs baseline: 1.1813x; 1.1813x over previous
import jax
import jax.numpy as jnp
from jax.experimental import pallas as pl


def _body(row_ref, col_ref, out_ref):
    h, out_n = row_ref.shape
    bblk = out_ref.shape[0]
    top = jnp.broadcast_to(row_ref[...][None, None, :, :], (bblk, h, h, out_n))
    bot = jnp.broadcast_to(col_ref[...][None, :, None, :], (bblk, h, h, out_n))
    out_ref[:, :, :, :out_n] = top
    out_ref[:, :, :, out_n:] = bot


def kernel(x, row_table, col_table):
    bs, _, h, w = x.shape
    out_n = row_table.shape[1]
    c_total = 2 * out_n
    bblk = 2

    y = pl.pallas_call(
        _body,
        grid=(bs // bblk,),
        in_specs=[
            pl.BlockSpec((h, out_n), lambda b: (0, 0)),
            pl.BlockSpec((w, out_n), lambda b: (0, 0)),
        ],
        out_specs=pl.BlockSpec((bblk, h, w, c_total), lambda b: (b, 0, 0, 0)),
        out_shape=jax.ShapeDtypeStruct((bs, h, w, c_total), jnp.float32),
    )(row_table, col_table)
    return jnp.transpose(y, (0, 3, 1, 2))
